# Initial kernel scaffold; baseline (speedup 1.0000x reference)
import jax
import jax.numpy as jnp
from jax.experimental import pallas as pl


def kernel(fea, idx_fea):
    raise NotImplementedError("write your pallas kernel here")


# speedup vs baseline: 3.6779x; 3.6779x over previous
import functools

import jax
import jax.numpy as jnp
from jax import lax
from jax.experimental import pallas as pl
from jax.experimental.pallas import tpu as pltpu
from jax.experimental.pallas import tpu_sc as plsc

N_EDGES = 320000
D_FEAT = 128
N_SEGMENTS = 10000

NC = 2
NS = 16
NW = NC * NS

EPT = N_EDGES // NW
CH = 80
NCHUNK = EPT // CH
SP = 10240
RPT = SP // NS
RCH = 128
NRCH = RPT // RCH


def _sc_segment_partials(fea, idx):
    mesh = plsc.VectorSubcoreMesh(core_axis_name="c", subcore_axis_name="s")

    @functools.partial(
        pl.kernel,
        mesh=mesh,
        out_type=jax.ShapeDtypeStruct((NC, SP, D_FEAT), jnp.float32),
        scratch_types=[
            pltpu.VMEM((CH, D_FEAT), jnp.float32),
            pltpu.VMEM((CH,), jnp.int32),
            pltpu.VMEM((RCH, D_FEAT), jnp.float32),
            pltpu.VMEM_SHARED((SP, D_FEAT), jnp.float32),
        ],
    )
    def k(fea_hbm, idx_hbm, part_hbm, rows_v, idx_v, zbuf, acc):
        cid = lax.axis_index("c")
        sid = lax.axis_index("s")
        wid = cid * NS + sid

        zvec = jnp.zeros((16,), jnp.float32)

        def zrow(r, carry):
            for j in range(D_FEAT // 16):
                zbuf[r, pl.ds(j * 16, 16)] = zvec
            return carry

        lax.fori_loop(0, RCH, zrow, 0)

        def zcopy(kk, carry):
            r0 = sid * RPT + kk * RCH
            pltpu.sync_copy(zbuf, acc.at[pl.ds(r0, RCH)])
            return carry

        lax.fori_loop(0, NRCH, zcopy, 0)
        plsc.subcore_barrier()

        base = wid * EPT

        def chunk(c, carry):
            e = base + c * CH
            pltpu.sync_copy(idx_hbm.at[pl.ds(e, CH)], idx_v)
            pltpu.sync_copy(fea_hbm.at[pl.ds(e, CH)], rows_v)
            pltpu.sync_copy(rows_v, acc.at[idx_v], add=True)
            return carry

        lax.fori_loop(0, NCHUNK, chunk, 0)
        plsc.subcore_barrier()

        def ocopy(kk, carry):
            r0 = sid * RPT + kk * RCH
            pltpu.sync_copy(acc.at[pl.ds(r0, RCH)], zbuf)
            pltpu.sync_copy(zbuf, part_hbm.at[cid, pl.ds(r0, RCH)])
            return carry

        lax.fori_loop(0, NRCH, ocopy, 0)

    return k(fea, idx)


def _tc_add(a, b):
    def add_kernel(a_ref, b_ref, o_ref):
        o_ref[...] = a_ref[...] + b_ref[...]

    blk = (1000, D_FEAT)
    return pl.pallas_call(
        add_kernel,
        grid=(N_SEGMENTS // blk[0],),
        in_specs=[
            pl.BlockSpec(blk, lambda i: (i, 0)),
            pl.BlockSpec(blk, lambda i: (i, 0)),
        ],
        out_specs=pl.BlockSpec(blk, lambda i: (i, 0)),
        out_shape=jax.ShapeDtypeStruct((N_SEGMENTS, D_FEAT), jnp.float32),
    )(a, b)


def kernel(fea, idx_fea):
    idx = idx_fea.astype(jnp.int32)
    part = _sc_segment_partials(fea, idx)
    return _tc_add(part[0], part[1])
